# baseline (speedup 1.0000x reference)
import functools

import jax
import jax.numpy as jnp
from jax import lax
from jax.experimental import pallas as pl
from jax.experimental.pallas import tpu as pltpu
from jax.experimental.pallas import tpu_sc as plsc

HIDDEN = 2048
NUM_EXPERTS = 16
TOP_K = 2
TOKENS = 16384
EPS = 1e-05
BLOCK = 512
CAST_BLOCK = 256
NEG = -3.0e38
GRID = TOKENS // BLOCK
CHUNKS = BLOCK // 16


def _cast_kernel(wtp_ref, out_ref):
    out_ref[...] = wtp_ref[...].astype(jnp.bfloat16)


def _logits_kernel(x_ref, t_ref, wtpb_ref, b_ref, wg_ref, o_ref):
    tcb = t_ref[...].astype(jnp.bfloat16)
    mm = lax.dot_general(tcb, wtpb_ref[...], (((1,), (1,)), ((), ())),
                         preferred_element_type=jnp.float32)
    xp = x_ref[...] + mm + b_ref[...]
    logits = lax.dot_general(
        wg_ref[...].astype(jnp.bfloat16), xp.astype(jnp.bfloat16),
        (((1,), (1,)), ((), ())), preferred_element_type=jnp.float32)
    o_ref[...] = logits.reshape(1, NUM_EXPERTS, BLOCK)


def _routing_body(lg_hbm, p1_hbm, p2_hbm, i1_hbm, i2_hbm,
                  lg_v, p1_v, p2_v, i1_v, i2_v):
    wid = lax.axis_index("s") * 2 + lax.axis_index("c")
    pltpu.sync_copy(lg_hbm.at[wid], lg_v)

    def chunk(c, carry):
        sl = pl.ds(c * 16, 16)
        ls = [lg_v[e, sl] for e in range(NUM_EXPERTS)]
        big = jnp.full((16,), NUM_EXPERTS, jnp.int32)
        neg = jnp.full((16,), NEG, jnp.float32)

        m1 = ls[0]
        for e in range(1, NUM_EXPERTS):
            m1 = jnp.maximum(m1, ls[e])
        i1 = big
        for e in range(NUM_EXPERTS):
            i1 = jnp.minimum(
                i1, jnp.where(ls[e] == m1, jnp.full((16,), e, jnp.int32),
                              big))
        l2s = [jnp.where(i1 == e, neg, ls[e]) for e in range(NUM_EXPERTS)]
        m2 = l2s[0]
        for e in range(1, NUM_EXPERTS):
            m2 = jnp.maximum(m2, l2s[e])
        i2 = big
        for e in range(NUM_EXPERTS):
            i2 = jnp.minimum(
                i2, jnp.where(l2s[e] == m2, jnp.full((16,), e, jnp.int32),
                              big))
        z = jnp.exp(ls[0] - m1)
        for e in range(1, NUM_EXPERTS):
            z = z + jnp.exp(ls[e] - m1)
        p1 = 1.0 / z
        p2 = jnp.exp(m2 - m1) / z
        s = p1 + p2 + EPS
        p1_v[sl] = p1 / s
        p2_v[sl] = p2 / s
        i1_v[sl] = i1
        i2_v[sl] = i2
        return carry

    lax.fori_loop(0, CHUNKS, chunk, 0)

    base = wid * BLOCK
    pltpu.sync_copy(p1_v, p1_hbm.at[pl.ds(base, BLOCK)])
    pltpu.sync_copy(p2_v, p2_hbm.at[pl.ds(base, BLOCK)])
    pltpu.sync_copy(i1_v, i1_hbm.at[pl.ds(base, BLOCK)])
    pltpu.sync_copy(i2_v, i2_hbm.at[pl.ds(base, BLOCK)])


def kernel(x, temporal_context, W_tp, b_tp, W_gate):
    wtpb = pl.pallas_call(
        _cast_kernel,
        grid=(HIDDEN // CAST_BLOCK,),
        in_specs=[pl.BlockSpec((CAST_BLOCK, HIDDEN), lambda i: (i, 0))],
        out_specs=pl.BlockSpec((CAST_BLOCK, HIDDEN), lambda i: (i, 0)),
        out_shape=jax.ShapeDtypeStruct((HIDDEN, HIDDEN), jnp.bfloat16),
    )(W_tp)

    logits3d = pl.pallas_call(
        _logits_kernel,
        grid=(GRID,),
        in_specs=[
            pl.BlockSpec((BLOCK, HIDDEN), lambda i: (i, 0)),
            pl.BlockSpec((BLOCK, HIDDEN), lambda i: (i, 0)),
            pl.BlockSpec((HIDDEN, HIDDEN), lambda i: (0, 0)),
            pl.BlockSpec((1, HIDDEN), lambda i: (0, 0)),
            pl.BlockSpec((NUM_EXPERTS, HIDDEN), lambda i: (0, 0)),
        ],
        out_specs=pl.BlockSpec((1, NUM_EXPERTS, BLOCK), lambda i: (i, 0, 0)),
        out_shape=jax.ShapeDtypeStruct((GRID, NUM_EXPERTS, BLOCK),
                                       jnp.float32),
    )(x, temporal_context, wtpb, b_tp.reshape(1, HIDDEN), W_gate)

    routing = functools.partial(
        pl.kernel,
        out_type=[
            jax.ShapeDtypeStruct((TOKENS,), jnp.float32),
            jax.ShapeDtypeStruct((TOKENS,), jnp.float32),
            jax.ShapeDtypeStruct((TOKENS,), jnp.int32),
            jax.ShapeDtypeStruct((TOKENS,), jnp.int32),
        ],
        mesh=plsc.VectorSubcoreMesh(core_axis_name="c", subcore_axis_name="s"),
        scratch_types=[
            pltpu.VMEM((NUM_EXPERTS, BLOCK), jnp.float32),
            pltpu.VMEM((BLOCK,), jnp.float32),
            pltpu.VMEM((BLOCK,), jnp.float32),
            pltpu.VMEM((BLOCK,), jnp.int32),
            pltpu.VMEM((BLOCK,), jnp.int32),
        ],
    )(_routing_body)
    p1, p2, i1, i2 = routing(logits3d)

    probs = jnp.stack([p1, p2], axis=1)
    idx = jnp.stack([i1, i2], axis=1)
    return probs, idx
